# Initial kernel scaffold; baseline (speedup 1.0000x reference)
import jax
import jax.numpy as jnp
from jax.experimental import pallas as pl


def kernel(sentence, emb_table, W, b):
    raise NotImplementedError("write your pallas kernel here")


# speedup vs baseline: 8.7141x; 8.7141x over previous
import functools

import jax
import jax.numpy as jnp
from jax import lax
from jax.experimental import pallas as pl
from jax.experimental.pallas import tpu as pltpu
from jax.experimental.pallas import tpu_sc as plsc

VOCAB = 100000
EMBED_DIM = 128
BATCH = 4096
HIST = 50

ROW_BLK = 8192
NUM_BLK = (VOCAB + ROW_BLK - 1) // ROW_BLK
S_LEN = NUM_BLK * ROW_BLK

NUM_WORKERS = 32
CHUNK = BATCH * HIST // NUM_WORKERS
VECS = CHUNK // 16
UNROLL = 8


def _scores_body(tab_ref, w_ref, b_ref, out_ref):
    out_ref[...] = (
        jnp.dot(tab_ref[...], w_ref[...], preferred_element_type=jnp.float32)
        + b_ref[0]
    )


def _scores(emb_table, w, b):
    return pl.pallas_call(
        _scores_body,
        grid=(NUM_BLK,),
        in_specs=[
            pl.BlockSpec((ROW_BLK, EMBED_DIM), lambda i: (i, 0)),
            pl.BlockSpec((EMBED_DIM, 1), lambda i: (0, 0)),
            pl.BlockSpec((1,), lambda i: (0,)),
        ],
        out_specs=pl.BlockSpec((ROW_BLK, 1), lambda i: (i, 0)),
        out_shape=jax.ShapeDtypeStruct((S_LEN, 1), jnp.float32),
    )(emb_table, w, b)


def _pool_body(s_hbm, sent_hbm, out_hbm, s_v, idx_v, acc_v):
    wid = lax.axis_index("s") * 2 + lax.axis_index("c")
    base = wid * CHUNK
    pltpu.sync_copy(sent_hbm.at[pl.ds(base, CHUNK)], idx_v)
    pltpu.sync_copy(s_hbm, s_v)
    for c in range(4):
        acc_v[pl.ds(c * 16, 16)] = jnp.zeros((16,), jnp.float32)

    def body(j, carry):
        for u in range(UNROLL):
            jj = j * UNROLL + u
            idx16 = idx_v[pl.ds(jj * 16, 16)]
            vals = plsc.load_gather(s_v, [idx16])
            col = (jj * 16 + lax.iota(jnp.int32, 16)) % HIST
            plsc.addupdate_scatter(acc_v, [col], vals)
        return carry

    lax.fori_loop(0, VECS // UNROLL, body, 0)
    pltpu.sync_copy(acc_v, out_hbm.at[wid])


_pool = pl.kernel(
    _pool_body,
    out_type=jax.ShapeDtypeStruct((NUM_WORKERS, 64), jnp.float32),
    mesh=plsc.VectorSubcoreMesh(core_axis_name="c", subcore_axis_name="s"),
    scratch_types=[
        pltpu.VMEM((S_LEN,), jnp.float32),
        pltpu.VMEM((CHUNK,), jnp.int32),
        pltpu.VMEM((64,), jnp.float32),
    ],
    compiler_params=pltpu.CompilerParams(needs_layout_passes=False),
)


def _finalize_body(p_ref, out_ref):
    tot = jnp.sum(p_ref[...], axis=0) * (1.0 / BATCH)
    out_ref[...] = jax.nn.sigmoid(tot)


def _finalize(partials):
    return pl.pallas_call(
        _finalize_body,
        out_shape=jax.ShapeDtypeStruct((64,), jnp.float32),
    )(partials)


def kernel(sentence, emb_table, W, b):
    s = _scores(emb_table, W.reshape(EMBED_DIM, 1), b).reshape(S_LEN)
    sent_flat = sentence.astype(jnp.int32).reshape(-1)
    partials = _pool(s, sent_flat)
    out64 = _finalize(partials)
    return out64[:HIST].reshape(HIST, 1)
